# baseline (speedup 1.0000x reference)
import functools

import jax
import jax.numpy as jnp
from jax import lax
from jax.experimental import pallas as pl
from jax.experimental.pallas import tpu as pltpu
from jax.experimental.pallas import tpu_sc as plsc


def _add_block(x_ref, emb_ref, o_ref):
    o_ref[...] = x_ref[...] + emb_ref[...]


def _kernel_tc(x, embed_weight):
    B, W, F = x.shape
    emb = embed_weight[:W]
    BB = 2
    return pl.pallas_call(
        _add_block,
        grid=(B // BB,),
        in_specs=[
            pl.BlockSpec((BB, W, F), lambda b: (b, 0, 0)),
            pl.BlockSpec((W, F), lambda b: (0, 0)),
        ],
        out_specs=pl.BlockSpec((BB, W, F), lambda b: (b, 0, 0)),
        out_shape=jax.ShapeDtypeStruct(x.shape, x.dtype),
        compiler_params=pltpu.CompilerParams(
            vmem_limit_bytes=100 * 1024 * 1024,
        ),
    )(x, emb)


def _kernel_sc(x, embed_weight):
    B, W, F = x.shape
    NW = 32
    BAND = W // NW
    NVEC = F // 16
    CH = 64
    CPB = BAND // CH
    M = B * CPB
    NBUF = 4
    T = M // NBUF

    mesh = plsc.VectorSubcoreMesh(core_axis_name="c", subcore_axis_name="s")

    @functools.partial(
        pl.kernel,
        mesh=mesh,
        out_type=jax.ShapeDtypeStruct((B, W, F), jnp.float32),
        scratch_types=[
            pltpu.VMEM((BAND, F), jnp.float32),
            pltpu.VMEM((NBUF, CH, F), jnp.float32),
            pltpu.VMEM((NBUF, CH, F), jnp.float32),
            pltpu.SemaphoreType.DMA,
        ]
        + [pltpu.SemaphoreType.DMA] * NBUF
        + [pltpu.SemaphoreType.DMA] * NBUF,
    )
    def sc_add(x_hbm, emb_hbm, out_hbm, emb_v, ibuf, obuf, sem_e, *sems):
        sem_in = sems[:NBUF]
        sem_out = sems[NBUF:]
        c = lax.axis_index("c")
        s = lax.axis_index("s")
        wid = s * 2 + c
        r0 = wid * BAND

        pltpu.make_async_copy(
            emb_hbm.at[pl.ds(r0, BAND)], emb_v, sem_e
        ).start()
        pltpu.make_async_copy(
            emb_hbm.at[pl.ds(r0, BAND)], emb_v, sem_e
        ).wait()

        def chunk_addr(m):
            b = m // CPB
            off = r0 + (m % CPB) * CH
            return b, off

        def start_in(m, k):
            b, off = chunk_addr(m)
            pltpu.make_async_copy(
                x_hbm.at[b, pl.ds(off, CH)], ibuf.at[k], sem_in[k]
            ).start()

        def wait_in(k):
            pltpu.make_async_copy(
                x_hbm.at[0, pl.ds(r0, CH)], ibuf.at[k], sem_in[k]
            ).wait()

        def start_out(m, k):
            b, off = chunk_addr(m)
            pltpu.make_async_copy(
                obuf.at[k], out_hbm.at[b, pl.ds(off, CH)], sem_out[k]
            ).start()

        def wait_out(k):
            pltpu.make_async_copy(
                obuf.at[k], out_hbm.at[0, pl.ds(r0, CH)], sem_out[k]
            ).wait()

        for k in range(NBUF):
            start_in(k, k)

        def body(t, carry):
            for k in range(NBUF):
                m = t * NBUF + k
                sub = m % CPB
                wait_in(k)

                @pl.when(t >= 1)
                def _():
                    wait_out(k)

                def row(r, carry2):
                    er = sub * CH + r
                    for l in range(NVEC):
                        sl = pl.ds(l * 16, 16)
                        obuf[k, r, sl] = ibuf[k, r, sl] + emb_v[er, sl]
                    return carry2

                lax.fori_loop(0, CH, row, 0)
                start_out(m, k)

                @pl.when(t + 1 < T)
                def _():
                    start_in(m + NBUF, k)
            return carry

        lax.fori_loop(0, T, body, 0)
        for k in range(NBUF):
            wait_out(k)

    return sc_add(x, embed_weight[:W])


def kernel(x, embed_weight):
    return _kernel_sc(x, embed_weight)


# speedup vs baseline: 1.0174x; 1.0174x over previous
import functools

import jax
import jax.numpy as jnp
from jax import lax
from jax.experimental import pallas as pl
from jax.experimental.pallas import tpu as pltpu
from jax.experimental.pallas import tpu_sc as plsc


def _add_block(x_ref, emb_ref, o_ref):
    o_ref[...] = x_ref[...] + emb_ref[...]


def _kernel_tc(x, embed_weight):
    B, W, F = x.shape
    emb = embed_weight[:W]
    BB = 2
    return pl.pallas_call(
        _add_block,
        grid=(B // BB,),
        in_specs=[
            pl.BlockSpec((BB, W, F), lambda b: (b, 0, 0)),
            pl.BlockSpec((W, F), lambda b: (0, 0)),
        ],
        out_specs=pl.BlockSpec((BB, W, F), lambda b: (b, 0, 0)),
        out_shape=jax.ShapeDtypeStruct(x.shape, x.dtype),
        compiler_params=pltpu.CompilerParams(
            vmem_limit_bytes=100 * 1024 * 1024,
        ),
    )(x, emb)


def _kernel_sc(x, embed_weight):
    B, W, F = x.shape
    NW = 32
    BAND = W // NW
    NVEC = F // 16
    CH = 64
    CPB = BAND // CH
    M = B * CPB
    NBUF = 4
    T = M // NBUF

    mesh = plsc.VectorSubcoreMesh(core_axis_name="c", subcore_axis_name="s")

    @functools.partial(
        pl.kernel,
        mesh=mesh,
        out_type=jax.ShapeDtypeStruct((B, W, F), jnp.float32),
        scratch_types=[
            pltpu.VMEM((BAND, F), jnp.float32),
            pltpu.VMEM((NBUF, CH, F), jnp.float32),
            pltpu.VMEM((NBUF, CH, F), jnp.float32),
            pltpu.SemaphoreType.DMA,
        ]
        + [pltpu.SemaphoreType.DMA] * NBUF
        + [pltpu.SemaphoreType.DMA] * NBUF,
    )
    def sc_add(x_hbm, emb_hbm, out_hbm, emb_v, ibuf, obuf, sem_e, *sems):
        sem_in = sems[:NBUF]
        sem_out = sems[NBUF:]
        c = lax.axis_index("c")
        s = lax.axis_index("s")
        wid = s * 2 + c
        r0 = wid * BAND

        pltpu.make_async_copy(
            emb_hbm.at[pl.ds(r0, BAND)], emb_v, sem_e
        ).start()
        pltpu.make_async_copy(
            emb_hbm.at[pl.ds(r0, BAND)], emb_v, sem_e
        ).wait()

        def chunk_addr(m):
            b = m // CPB
            off = r0 + (m % CPB) * CH
            return b, off

        def start_in(m, k):
            b, off = chunk_addr(m)
            pltpu.make_async_copy(
                x_hbm.at[b, pl.ds(off, CH)], ibuf.at[k], sem_in[k]
            ).start()

        def wait_in(k):
            pltpu.make_async_copy(
                x_hbm.at[0, pl.ds(r0, CH)], ibuf.at[k], sem_in[k]
            ).wait()

        def start_out(m, k):
            b, off = chunk_addr(m)
            pltpu.make_async_copy(
                obuf.at[k], out_hbm.at[b, pl.ds(off, CH)], sem_out[k]
            ).start()

        def wait_out(k):
            pltpu.make_async_copy(
                obuf.at[k], out_hbm.at[0, pl.ds(r0, CH)], sem_out[k]
            ).wait()

        for k in range(NBUF):
            start_in(k, k)

        def body(t, carry):
            for k in range(NBUF):
                m = t * NBUF + k
                sub = m % CPB
                wait_in(k)

                @pl.when(t >= 1)
                def _():
                    wait_out(k)

                def row(r, carry2):
                    er = sub * CH + r
                    for l in range(NVEC):
                        sl = pl.ds(l * 16, 16)
                        obuf[k, r, sl] = ibuf[k, r, sl] + emb_v[er, sl]
                    return carry2

                lax.fori_loop(0, CH, row, 0)
                start_out(m, k)

                @pl.when(t + 1 < T)
                def _():
                    start_in(m + NBUF, k)
            return carry

        lax.fori_loop(0, T, body, 0)
        for k in range(NBUF):
            wait_out(k)

    return sc_add(x, embed_weight[:W])


def _kernel_sc2(x, embed_weight):
    B, W, F = x.shape
    NW = 32
    BAND = W // NW
    NVEC = F // 16
    CH = 2
    M = BAND // CH
    NBUF = 4
    T = M // NBUF

    mesh = plsc.VectorSubcoreMesh(core_axis_name="c", subcore_axis_name="s")

    @functools.partial(
        pl.kernel,
        mesh=mesh,
        out_type=jax.ShapeDtypeStruct((B, W, F), jnp.float32),
        scratch_types=[
            pltpu.VMEM((NBUF, CH, F), jnp.float32),
            pltpu.VMEM((NBUF, B, CH, F), jnp.float32),
            pltpu.VMEM((NBUF, B, CH, F), jnp.float32),
        ]
        + [pltpu.SemaphoreType.DMA] * NBUF
        + [pltpu.SemaphoreType.DMA] * NBUF
        + [pltpu.SemaphoreType.DMA] * NBUF,
    )
    def sc_add(x_hbm, emb_hbm, out_hbm, embbuf, ibuf, obuf, *sems):
        sem_emb = sems[:NBUF]
        sem_in = sems[NBUF : 2 * NBUF]
        sem_out = sems[2 * NBUF :]
        c = lax.axis_index("c")
        s = lax.axis_index("s")
        wid = s * 2 + c
        r0 = wid * BAND

        def start_in(m, k):
            off = r0 + m * CH
            pltpu.make_async_copy(
                x_hbm.at[:, pl.ds(off, CH)], ibuf.at[k], sem_in[k]
            ).start()
            pltpu.make_async_copy(
                emb_hbm.at[pl.ds(off, CH)], embbuf.at[k], sem_emb[k]
            ).start()

        def wait_in(k):
            pltpu.make_async_copy(
                x_hbm.at[:, pl.ds(r0, CH)], ibuf.at[k], sem_in[k]
            ).wait()
            pltpu.make_async_copy(
                emb_hbm.at[pl.ds(r0, CH)], embbuf.at[k], sem_emb[k]
            ).wait()

        def start_out(m, k):
            off = r0 + m * CH
            pltpu.make_async_copy(
                obuf.at[k], out_hbm.at[:, pl.ds(off, CH)], sem_out[k]
            ).start()

        def wait_out(k):
            pltpu.make_async_copy(
                obuf.at[k], out_hbm.at[:, pl.ds(r0, CH)], sem_out[k]
            ).wait()

        for k in range(NBUF):
            start_in(k, k)

        def body(t, carry):
            for k in range(NBUF):
                m = t * NBUF + k
                wait_in(k)

                @pl.when(t >= 1)
                def _():
                    wait_out(k)

                evs = tuple(
                    embbuf[k, r, pl.ds(l * 16, 16)]
                    for r in range(CH)
                    for l in range(NVEC)
                )

                def bbody(b, ev):
                    idx = 0
                    for r in range(CH):
                        for l in range(NVEC):
                            sl = pl.ds(l * 16, 16)
                            obuf[k, b, r, sl] = ibuf[k, b, r, sl] + ev[idx]
                            idx += 1
                    return ev

                lax.fori_loop(0, B, bbody, evs)
                start_out(m, k)

                @pl.when(t + 1 < T)
                def _():
                    start_in(m + NBUF, k)
            return carry

        lax.fori_loop(0, T, body, 0)
        for k in range(NBUF):
            wait_out(k)

    return sc_add(x, embed_weight[:W])


def kernel(x, embed_weight):
    return _kernel_sc2(x, embed_weight)


# speedup vs baseline: 1.3466x; 1.3236x over previous
import functools

import jax
import jax.numpy as jnp
from jax import lax
from jax.experimental import pallas as pl
from jax.experimental.pallas import tpu as pltpu
from jax.experimental.pallas import tpu_sc as plsc


def _add_block(x_ref, emb_ref, o_ref):
    o_ref[...] = x_ref[...] + emb_ref[...]


def _kernel_tc(x, embed_weight):
    B, W, F = x.shape
    emb = embed_weight[:W]
    BB = 4
    WS = W // 2
    return pl.pallas_call(
        _add_block,
        grid=(2, B // BB),
        in_specs=[
            pl.BlockSpec((BB, WS, F), lambda j, b: (b, j, 0)),
            pl.BlockSpec((WS, F), lambda j, b: (j, 0)),
        ],
        out_specs=pl.BlockSpec((BB, WS, F), lambda j, b: (b, j, 0)),
        out_shape=jax.ShapeDtypeStruct(x.shape, x.dtype),
        compiler_params=pltpu.CompilerParams(
            vmem_limit_bytes=100 * 1024 * 1024,
        ),
    )(x, emb)


def _kernel_sc(x, embed_weight):
    B, W, F = x.shape
    NW = 32
    BAND = W // NW
    NVEC = F // 16
    CH = 64
    CPB = BAND // CH
    M = B * CPB
    NBUF = 4
    T = M // NBUF

    mesh = plsc.VectorSubcoreMesh(core_axis_name="c", subcore_axis_name="s")

    @functools.partial(
        pl.kernel,
        mesh=mesh,
        out_type=jax.ShapeDtypeStruct((B, W, F), jnp.float32),
        scratch_types=[
            pltpu.VMEM((BAND, F), jnp.float32),
            pltpu.VMEM((NBUF, CH, F), jnp.float32),
            pltpu.VMEM((NBUF, CH, F), jnp.float32),
            pltpu.SemaphoreType.DMA,
        ]
        + [pltpu.SemaphoreType.DMA] * NBUF
        + [pltpu.SemaphoreType.DMA] * NBUF,
    )
    def sc_add(x_hbm, emb_hbm, out_hbm, emb_v, ibuf, obuf, sem_e, *sems):
        sem_in = sems[:NBUF]
        sem_out = sems[NBUF:]
        c = lax.axis_index("c")
        s = lax.axis_index("s")
        wid = s * 2 + c
        r0 = wid * BAND

        pltpu.make_async_copy(
            emb_hbm.at[pl.ds(r0, BAND)], emb_v, sem_e
        ).start()
        pltpu.make_async_copy(
            emb_hbm.at[pl.ds(r0, BAND)], emb_v, sem_e
        ).wait()

        def chunk_addr(m):
            b = m // CPB
            off = r0 + (m % CPB) * CH
            return b, off

        def start_in(m, k):
            b, off = chunk_addr(m)
            pltpu.make_async_copy(
                x_hbm.at[b, pl.ds(off, CH)], ibuf.at[k], sem_in[k]
            ).start()

        def wait_in(k):
            pltpu.make_async_copy(
                x_hbm.at[0, pl.ds(r0, CH)], ibuf.at[k], sem_in[k]
            ).wait()

        def start_out(m, k):
            b, off = chunk_addr(m)
            pltpu.make_async_copy(
                obuf.at[k], out_hbm.at[b, pl.ds(off, CH)], sem_out[k]
            ).start()

        def wait_out(k):
            pltpu.make_async_copy(
                obuf.at[k], out_hbm.at[0, pl.ds(r0, CH)], sem_out[k]
            ).wait()

        for k in range(NBUF):
            start_in(k, k)

        def body(t, carry):
            for k in range(NBUF):
                m = t * NBUF + k
                sub = m % CPB
                wait_in(k)

                @pl.when(t >= 1)
                def _():
                    wait_out(k)

                def row(r, carry2):
                    er = sub * CH + r
                    for l in range(NVEC):
                        sl = pl.ds(l * 16, 16)
                        obuf[k, r, sl] = ibuf[k, r, sl] + emb_v[er, sl]
                    return carry2

                lax.fori_loop(0, CH, row, 0)
                start_out(m, k)

                @pl.when(t + 1 < T)
                def _():
                    start_in(m + NBUF, k)
            return carry

        lax.fori_loop(0, T, body, 0)
        for k in range(NBUF):
            wait_out(k)

    return sc_add(x, embed_weight[:W])


def _kernel_sc2(x, embed_weight):
    B, W, F = x.shape
    NW = 32
    BAND = W // NW
    NVEC = F // 16
    CH = 2
    M = BAND // CH
    NBUF = 4
    T = M // NBUF

    mesh = plsc.VectorSubcoreMesh(core_axis_name="c", subcore_axis_name="s")

    @functools.partial(
        pl.kernel,
        mesh=mesh,
        out_type=jax.ShapeDtypeStruct((B, W, F), jnp.float32),
        scratch_types=[
            pltpu.VMEM((NBUF, CH, F), jnp.float32),
            pltpu.VMEM((NBUF, B, CH, F), jnp.float32),
            pltpu.VMEM((NBUF, B, CH, F), jnp.float32),
        ]
        + [pltpu.SemaphoreType.DMA] * NBUF
        + [pltpu.SemaphoreType.DMA] * NBUF
        + [pltpu.SemaphoreType.DMA] * NBUF,
    )
    def sc_add(x_hbm, emb_hbm, out_hbm, embbuf, ibuf, obuf, *sems):
        sem_emb = sems[:NBUF]
        sem_in = sems[NBUF : 2 * NBUF]
        sem_out = sems[2 * NBUF :]
        c = lax.axis_index("c")
        s = lax.axis_index("s")
        wid = s * 2 + c
        r0 = wid * BAND

        def start_in(m, k):
            off = r0 + m * CH
            pltpu.make_async_copy(
                x_hbm.at[:, pl.ds(off, CH)], ibuf.at[k], sem_in[k]
            ).start()
            pltpu.make_async_copy(
                emb_hbm.at[pl.ds(off, CH)], embbuf.at[k], sem_emb[k]
            ).start()

        def wait_in(k):
            pltpu.make_async_copy(
                x_hbm.at[:, pl.ds(r0, CH)], ibuf.at[k], sem_in[k]
            ).wait()
            pltpu.make_async_copy(
                emb_hbm.at[pl.ds(r0, CH)], embbuf.at[k], sem_emb[k]
            ).wait()

        def start_out(m, k):
            off = r0 + m * CH
            pltpu.make_async_copy(
                obuf.at[k], out_hbm.at[:, pl.ds(off, CH)], sem_out[k]
            ).start()

        def wait_out(k):
            pltpu.make_async_copy(
                obuf.at[k], out_hbm.at[:, pl.ds(r0, CH)], sem_out[k]
            ).wait()

        for k in range(NBUF):
            start_in(k, k)

        def body(t, carry):
            for k in range(NBUF):
                m = t * NBUF + k
                wait_in(k)

                @pl.when(t >= 1)
                def _():
                    wait_out(k)

                evs = tuple(
                    embbuf[k, r, pl.ds(l * 16, 16)]
                    for r in range(CH)
                    for l in range(NVEC)
                )

                def bbody(b, ev):
                    idx = 0
                    for r in range(CH):
                        for l in range(NVEC):
                            sl = pl.ds(l * 16, 16)
                            obuf[k, b, r, sl] = ibuf[k, b, r, sl] + ev[idx]
                            idx += 1
                    return ev

                lax.fori_loop(0, B, bbody, evs)
                start_out(m, k)

                @pl.when(t + 1 < T)
                def _():
                    start_in(m + NBUF, k)
            return carry

        lax.fori_loop(0, T, body, 0)
        for k in range(NBUF):
            wait_out(k)

    return sc_add(x, embed_weight[:W])


def kernel(x, embed_weight):
    return _kernel_tc(x, embed_weight)


# speedup vs baseline: 1.3571x; 1.0078x over previous
import jax
import jax.numpy as jnp
from jax.experimental import pallas as pl
from jax.experimental.pallas import tpu as pltpu


def _add_block(x_ref, emb_ref, o_ref):
    o_ref[...] = x_ref[...] + emb_ref[...]


def kernel(x, embed_weight):
    B, W, F = x.shape
    emb = embed_weight[:W]
    BB = 2
    return pl.pallas_call(
        _add_block,
        grid=(B // BB,),
        in_specs=[
            pl.BlockSpec((BB, W, F), lambda b: (b, 0, 0)),
            pl.BlockSpec((W, F), lambda b: (0, 0)),
        ],
        out_specs=pl.BlockSpec((BB, W, F), lambda b: (b, 0, 0)),
        out_shape=jax.ShapeDtypeStruct(x.shape, x.dtype),
        compiler_params=pltpu.CompilerParams(
            vmem_limit_bytes=100 * 1024 * 1024,
        ),
    )(x, emb)
